# baseline (speedup 1.0000x reference)
import jax
import jax.numpy as jnp
from jax.experimental import pallas as pl
from jax.experimental.pallas import tpu as pltpu

_N = 200000
_STRIDE = 3000
_NIDS = 64
_NBUF = 4
_SLOT_ROWS = 16000

_CHUNK_SIZES = (
    [1000, 2000, 4000, 8000, 16000]
    + [13000] * 26
    + [16000, 8000, 4000, 2000, 1000]
)
assert sum(_CHUNK_SIZES) == 2 * _N
assert max(_CHUNK_SIZES) <= _SLOT_ROWS

_CHUNKS = []
_pos = 0
for _c in _CHUNK_SIZES:
    _slab, _start = divmod(_pos, _N)
    assert _start + _c <= _N
    _zr = []
    if _slab == 0:
        _k0 = -(-_start // _STRIDE)
        _k1 = (_start + _c - 1) // _STRIDE
        _zr = [_k * _STRIDE - _start
               for _k in range(_k0, min(_k1, _NIDS - 1) + 1)]
    _CHUNKS.append((_slab, _start, _c, _zr))
    _pos += _c
_NCHUNKS = len(_CHUNKS)


def _stream_kernel(x_ref, o_ref, b0, b1, b2, b3, in_sems, out_sems):
    bufs = (b0, b1, b2, b3)
    zrow = jnp.zeros((128,), jnp.float32)

    def make_in(k):
        slab, start, c, _ = _CHUNKS[k]
        return pltpu.make_async_copy(
            x_ref.at[slab, pl.ds(start, c), :],
            bufs[k % _NBUF].at[pl.ds(0, c), :],
            in_sems.at[k])

    def make_out(k):
        slab, start, c, _ = _CHUNKS[k]
        return pltpu.make_async_copy(
            bufs[k % _NBUF].at[pl.ds(0, c), :],
            o_ref.at[slab, pl.ds(start, c), :],
            out_sems.at[k])

    ins = {}
    outs = {}
    for k in range(_NBUF):
        ins[k] = make_in(k)
        ins[k].start()
    for k in range(_NCHUNKS):
        ins[k].wait()
        for r in _CHUNKS[k][3]:
            bufs[k % _NBUF][r, :] = zrow
        outs[k] = make_out(k)
        outs[k].start()
        if k + _NBUF < _NCHUNKS:
            outs[k].wait()
            ins[k + _NBUF] = make_in(k + _NBUF)
            ins[k + _NBUF].start()
    for k in range(_NCHUNKS - _NBUF, _NCHUNKS):
        outs[k].wait()


def kernel(data):
    slot = pltpu.MemorySpace.VMEM((_SLOT_ROWS, 128), jnp.float32)
    return pl.pallas_call(
        _stream_kernel,
        in_specs=[pl.BlockSpec(memory_space=pltpu.MemorySpace.HBM)],
        out_specs=pl.BlockSpec(memory_space=pltpu.MemorySpace.HBM),
        out_shape=jax.ShapeDtypeStruct(data.shape, data.dtype),
        scratch_shapes=[
            slot, slot, slot, slot,
            pltpu.SemaphoreType.DMA((_NCHUNKS,)),
            pltpu.SemaphoreType.DMA((_NCHUNKS,)),
        ],
    )(data)


# speedup vs baseline: 1.0243x; 1.0243x over previous
import jax
import jax.numpy as jnp
from jax.experimental import pallas as pl
from jax.experimental.pallas import tpu as pltpu

_N = 200000
_STRIDE = 3000
_NIDS = 64
_NBUF = 6
_LAG = 2
_SLOT_ROWS = 16000

_CHUNK_SIZES = (
    [1000, 2000, 4000, 8000, 16000]
    + [13000] * 26
    + [16000, 8000, 4000, 2000, 1000]
)
assert sum(_CHUNK_SIZES) == 2 * _N
assert max(_CHUNK_SIZES) <= _SLOT_ROWS

_CHUNKS = []
_pos = 0
for _c in _CHUNK_SIZES:
    _slab, _start = divmod(_pos, _N)
    assert _start + _c <= _N
    _zr = []
    if _slab == 0:
        _k0 = -(-_start // _STRIDE)
        _k1 = (_start + _c - 1) // _STRIDE
        _zr = [_k * _STRIDE - _start
               for _k in range(_k0, min(_k1, _NIDS - 1) + 1)]
    _CHUNKS.append((_slab, _start, _c, _zr))
    _pos += _c
_NCHUNKS = len(_CHUNKS)


def _stream_kernel(x_ref, o_ref, b0, b1, b2, b3, b4, b5, in_sems, out_sems):
    bufs = (b0, b1, b2, b3, b4, b5)
    zrow = jnp.zeros((128,), jnp.float32)

    def make_in(k):
        slab, start, c, _ = _CHUNKS[k]
        return pltpu.make_async_copy(
            x_ref.at[slab, pl.ds(start, c), :],
            bufs[k % _NBUF].at[pl.ds(0, c), :],
            in_sems.at[k])

    def make_out(k):
        slab, start, c, _ = _CHUNKS[k]
        return pltpu.make_async_copy(
            bufs[k % _NBUF].at[pl.ds(0, c), :],
            o_ref.at[slab, pl.ds(start, c), :],
            out_sems.at[k])

    depth = _NBUF - _LAG
    ins = {}
    outs = {}
    waited = set()
    for k in range(min(depth, _NCHUNKS)):
        ins[k] = make_in(k)
        ins[k].start()
    for k in range(_NCHUNKS):
        ins[k].wait()
        for r in _CHUNKS[k][3]:
            bufs[k % _NBUF][r, :] = zrow
        outs[k] = make_out(k)
        outs[k].start()
        nxt = k + depth
        if nxt < _NCHUNKS:
            prev = nxt - _NBUF
            if prev >= 0:
                outs[prev].wait()
                waited.add(prev)
            ins[nxt] = make_in(nxt)
            ins[nxt].start()
    for k in range(_NCHUNKS):
        if k not in waited:
            outs[k].wait()


def kernel(data):
    slot = pltpu.MemorySpace.VMEM((_SLOT_ROWS, 128), jnp.float32)
    return pl.pallas_call(
        _stream_kernel,
        in_specs=[pl.BlockSpec(memory_space=pltpu.MemorySpace.HBM)],
        out_specs=pl.BlockSpec(memory_space=pltpu.MemorySpace.HBM),
        out_shape=jax.ShapeDtypeStruct(data.shape, data.dtype),
        scratch_shapes=[
            slot, slot, slot, slot, slot, slot,
            pltpu.SemaphoreType.DMA((_NCHUNKS,)),
            pltpu.SemaphoreType.DMA((_NCHUNKS,)),
        ],
    )(data)


# speedup vs baseline: 1.0258x; 1.0015x over previous
import jax
import jax.numpy as jnp
from jax.experimental import pallas as pl

_B = 25000
_STRIDE = 3000
_NIDS = 64


def _copy_kernel(x_ref, o_ref):
    i = pl.program_id(0)
    j = pl.program_id(1)
    o_ref[0] = x_ref[0]
    for k in range(_NIDS):
        rid = k * _STRIDE

        @pl.when((i == 0) & (j == rid // _B))
        def _zero_row(rid=rid):
            o_ref[0, rid % _B, :] = jnp.zeros((128,), jnp.float32)


def kernel(data):
    n = data.shape[1]
    return pl.pallas_call(
        _copy_kernel,
        grid=(data.shape[0], n // _B),
        in_specs=[pl.BlockSpec((1, _B, 128), lambda i, j: (i, j, 0))],
        out_specs=pl.BlockSpec((1, _B, 128), lambda i, j: (i, j, 0)),
        out_shape=jax.ShapeDtypeStruct(data.shape, data.dtype),
    )(data)
